# baseline (speedup 1.0000x reference)
import dataclasses
import functools

import jax
import jax.numpy as jnp
from jax import lax
from jax.experimental import pallas as pl
from jax.experimental.pallas import tpu as pltpu
from jax.experimental.pallas import tpu_sc as plsc

D = 128
L = 16
NC = 2
NS = 16
NW = NC * NS
SCHUNK = 8
NJ = D // L


def _bcast_last(v):
    idx = lax.full((L,), L - 1, jnp.int32)
    dnums = lax.GatherDimensionNumbers(
        offset_dims=(), collapsed_slice_dims=(0,), start_index_map=(0,))
    return lax.gather(v, idx[:, None], dnums, slice_sizes=(1,),
                      mode=lax.GatherScatterMode.PROMISE_IN_BOUNDS)


def _ln_token(rows_v, pos_v, b, i, p_row, gs, bs):
    acc = jnp.zeros((L,), jnp.float32)
    acc2 = jnp.zeros((L,), jnp.float32)
    vs = []
    for j in range(NJ):
        v = rows_v[b, i, pl.ds(j * L, L)] + pos_v[p_row, pl.ds(j * L, L)]
        vs.append(v)
        acc = acc + v
        acc2 = acc2 + v * v
    mv = _bcast_last(jnp.cumsum(acc)) * (1.0 / D)
    s2v = _bcast_last(jnp.cumsum(acc2)) * (1.0 / D)
    xv = s2v - mv * mv + 1e-5
    bits = lax.bitcast_convert_type(xv, jnp.int32)
    bits = 0x5F3759DF - lax.shift_right_arithmetic(bits, 1)
    y = lax.bitcast_convert_type(bits, jnp.float32)
    for _ in range(3):
        y = y * (1.5 - 0.5 * xv * y * y)
    for j in range(NJ):
        rows_v[b, i, pl.ds(j * L, L)] = (vs[j] - mv) * y * gs[j] + bs[j]


def kernel(x, table, pos, gamma, beta):
    B, S = x.shape
    s_per_w = S // NW
    n_chunks = s_per_w // SCHUNK

    mesh = plsc.VectorSubcoreMesh(core_axis_name="c", subcore_axis_name="s")
    cp = pltpu.CompilerParams()
    if "needs_layout_passes" in pltpu.CompilerParams.__dataclass_fields__:
        cp = dataclasses.replace(cp, needs_layout_passes=False)

    vmem = pltpu.VMEM

    @functools.partial(
        pl.kernel,
        mesh=mesh,
        out_type=jax.ShapeDtypeStruct((B, S, D), jnp.float32),
        scratch_types=[
            vmem((B, S // NW), jnp.int32),
            vmem((2, B, SCHUNK, D), jnp.float32),
            vmem((s_per_w, D), jnp.float32),
            vmem((D,), jnp.float32),
            vmem((D,), jnp.float32),
            pltpu.SemaphoreType.DMA,
            pltpu.SemaphoreType.DMA,
            pltpu.SemaphoreType.DMA,
            pltpu.SemaphoreType.DMA,
        ],
        compiler_params=cp,
    )
    def sc_embed(x_hbm, tab_hbm, pos_hbm, g_hbm, b_hbm, out_hbm,
                 idx_v, rows_v, pos_v, g_v, b_v, sg0, sg1, so0, so1):
        wid = lax.axis_index("s") * NC + lax.axis_index("c")
        s_base = wid * s_per_w
        pltpu.sync_copy(g_hbm, g_v)
        pltpu.sync_copy(b_hbm, b_v)
        pltpu.sync_copy(pos_hbm.at[pl.ds(s_base, s_per_w)], pos_v)
        for b in range(B):
            pltpu.sync_copy(x_hbm.at[pl.ds(b * S + s_base, s_per_w)],
                            idx_v.at[b])
        gs = [g_v[pl.ds(j * L, L)] for j in range(NJ)]
        bs = [b_v[pl.ds(j * L, L)] for j in range(NJ)]
        sg = [sg0, sg1]
        so = [so0, so1]

        def issue_gather(ci, buf):
            return [
                pltpu.async_copy(
                    tab_hbm.at[idx_v.at[b, pl.ds(ci * SCHUNK, SCHUNK)]],
                    rows_v.at[buf, b], sg[buf])
                for b in range(B)
            ]

        gathers = {0: issue_gather(0, 0)}
        out_copies = {}
        for ci in range(n_chunks):
            cur = ci % 2
            if ci + 1 < n_chunks:
                if ci - 1 in out_copies:
                    out_copies.pop(ci - 1).wait()
                gathers[ci + 1] = issue_gather(ci + 1, 1 - cur)
            for g in gathers.pop(ci):
                g.wait()

            @plsc.parallel_loop(0, B * SCHUNK, 1, unroll=2)
            def _(r):
                b = lax.shift_right_logical(r, 3)
                i = lax.bitwise_and(r, SCHUNK - 1)
                _ln_token(rows_v.at[cur], pos_v, b, i, ci * SCHUNK + i, gs, bs)

            s_off = s_base + ci * SCHUNK
            out_copies[ci] = pltpu.async_copy(
                rows_v.at[cur], out_hbm.at[:, pl.ds(s_off, SCHUNK), :], so[cur])
        for c in out_copies.values():
            c.wait()

    return sc_embed(x.reshape(B * S), table, pos, gamma, beta)
